# baseline (device time: 42103 ns/iter reference)
import jax
import jax.numpy as jnp
from jax import lax
from jax.experimental import pallas as pl
from jax.experimental.pallas import tpu as pltpu


def kernel(Q, K, V):
    b, s, h, d = Q.shape
    scale = d ** -0.5
    sb = s // 4

    Qt = jnp.transpose(Q, (0, 2, 1, 3))
    Kt = jnp.transpose(K, (0, 2, 1, 3))
    Vt = jnp.transpose(V, (0, 2, 1, 3))

    def body(qt, kt, vt, out_t,
             qb_rem, ou_loc, ou_send, ou_rem, st_loc, st_send, st_rem, fb,
             send_sems, recv_sems):
        mx = lax.axis_index("x")
        my = lax.axis_index("y")
        mz = lax.axis_index("z")
        ynbr = (mx, 1 - my, mz)
        xnbr = (1 - mx, my, mz)
        znbr = (mx, my, 1 - mz)
        dnbr = (1 - mx, my, 1 - mz)
        p = 2 * mx + mz
        qoff = sb * p

        barrier_sem = pltpu.get_barrier_semaphore()
        for nbr in (ynbr, xnbr, znbr, dnbr):
            pl.semaphore_signal(barrier_sem, inc=1, device_id=nbr,
                                device_id_type=pl.DeviceIdType.MESH)
        pl.semaphore_wait(barrier_sem, 4)

        r_q = pltpu.make_async_remote_copy(
            src_ref=qt.at[:, :, pl.ds(qoff, sb), :], dst_ref=qb_rem,
            send_sem=send_sems.at[0], recv_sem=recv_sems.at[0],
            device_id=ynbr, device_id_type=pl.DeviceIdType.MESH)
        r_q.start()

        def partial_attn(get_q, ou, st):
            for bi in range(b):
                for hi in range(h):
                    q = get_q(bi, hi) * scale
                    k = kt[bi, hi]
                    sc = lax.dot_general(q, k, (((1,), (1,)), ((), ())),
                                         preferred_element_type=jnp.float32)
                    m = jnp.max(sc, axis=1, keepdims=True)
                    e = jnp.exp(sc - m)
                    l = jnp.sum(e, axis=1, keepdims=True)
                    o = lax.dot_general(e, vt[bi, hi], (((1,), (0,)), ((), ())),
                                        preferred_element_type=jnp.float32)
                    ou[bi, hi] = o
                    st[0, bi, hi] = m[:, 0]
                    st[1, bi, hi] = l[:, 0]

        partial_attn(lambda bi, hi: qt[bi, hi, pl.ds(qoff, sb), :],
                     ou_loc, st_loc)

        r_q.wait()

        partial_attn(lambda bi, hi: qb_rem[bi, hi], ou_send, st_send)

        r_o = pltpu.make_async_remote_copy(
            src_ref=ou_send, dst_ref=ou_rem,
            send_sem=send_sems.at[1], recv_sem=recv_sems.at[1],
            device_id=ynbr, device_id_type=pl.DeviceIdType.MESH)
        r_s = pltpu.make_async_remote_copy(
            src_ref=st_send, dst_ref=st_rem,
            send_sem=send_sems.at[2], recv_sem=recv_sems.at[2],
            device_id=ynbr, device_id_type=pl.DeviceIdType.MESH)
        r_o.start()
        r_s.start()
        r_o.wait()
        r_s.wait()

        for bi in range(b):
            for hi in range(h):
                m1 = st_loc[0, bi, hi].reshape(sb, 1)
                l1 = st_loc[1, bi, hi].reshape(sb, 1)
                m2 = st_rem[0, bi, hi].reshape(sb, 1)
                l2 = st_rem[1, bi, hi].reshape(sb, 1)
                m = jnp.maximum(m1, m2)
                a1 = jnp.exp(m1 - m)
                a2 = jnp.exp(m2 - m)
                o = a1 * ou_loc[bi, hi] + a2 * ou_rem[bi, hi]
                res = o / (a1 * l1 + a2 * l2)
                fb[bi, hi] = res
                out_t[bi, hi, pl.ds(qoff, sb), :] = res

        dist = []
        for j, nbr in enumerate((xnbr, znbr, dnbr)):
            r = pltpu.make_async_remote_copy(
                src_ref=fb, dst_ref=out_t.at[:, :, pl.ds(qoff, sb), :],
                send_sem=send_sems.at[3 + j], recv_sem=recv_sems.at[3 + j],
                device_id=nbr, device_id_type=pl.DeviceIdType.MESH)
            r.start()
            dist.append(r)
        for r in dist:
            r.wait()

    out_t = pl.pallas_call(
        body,
        out_shape=jax.ShapeDtypeStruct((b, h, s, d), jnp.float32),
        in_specs=[pl.BlockSpec(memory_space=pltpu.VMEM)] * 3,
        out_specs=pl.BlockSpec(memory_space=pltpu.VMEM),
        scratch_shapes=[
            pltpu.VMEM((b, h, sb, d), jnp.float32),
            pltpu.VMEM((b, h, sb, d), jnp.float32),
            pltpu.VMEM((b, h, sb, d), jnp.float32),
            pltpu.VMEM((b, h, sb, d), jnp.float32),
            pltpu.VMEM((2, b, h, sb), jnp.float32),
            pltpu.VMEM((2, b, h, sb), jnp.float32),
            pltpu.VMEM((2, b, h, sb), jnp.float32),
            pltpu.VMEM((b, h, sb, d), jnp.float32),
            pltpu.SemaphoreType.DMA((6,)),
            pltpu.SemaphoreType.DMA((6,)),
        ],
        compiler_params=pltpu.CompilerParams(collective_id=0),
    )(Qt, Kt, Vt)
    return jnp.transpose(out_t, (0, 2, 1, 3))


# device time: 31738 ns/iter; 1.3266x vs baseline; 1.3266x over previous
import jax
import jax.numpy as jnp
from jax import lax
from jax.experimental import pallas as pl
from jax.experimental.pallas import tpu as pltpu

NSUB = 4


def kernel(Q, K, V):
    b, s, h, d = Q.shape
    scale = d ** -0.5
    sb = s // 4
    hh = h // 2

    Qt = jnp.transpose(Q, (0, 2, 1, 3))
    Kt = jnp.transpose(K, (0, 2, 1, 3))
    Vt = jnp.transpose(V, (0, 2, 1, 3))

    subs = [(bi, hf) for bi in range(b) for hf in range(2)]

    def body(qt, kt, vt, out_t,
             qb_send, qb_rem, pl_loc, pl_send, pl_rem, fb,
             qs_sem, qr_sem, os_sem, or_sem, ds_sem, dr_sem):
        mx = lax.axis_index("x")
        my = lax.axis_index("y")
        mz = lax.axis_index("z")
        ynbr = (mx, 1 - my, mz)
        xnbr = (1 - mx, my, mz)
        znbr = (mx, my, 1 - mz)
        dnbr = (1 - mx, my, 1 - mz)
        qoff = sb * (2 * mx + mz)

        barrier_sem = pltpu.get_barrier_semaphore()
        for nbr in (ynbr, xnbr, znbr, dnbr):
            pl.semaphore_signal(barrier_sem, inc=1, device_id=nbr,
                                device_id_type=pl.DeviceIdType.MESH)
        pl.semaphore_wait(barrier_sem, 4)

        for bi in range(b):
            for hi in range(h):
                qb_send[bi, hi] = qt[bi, hi, pl.ds(qoff, sb), :]

        r_q = []
        for j, (bi, hf) in enumerate(subs):
            r = pltpu.make_async_remote_copy(
                src_ref=qb_send.at[bi, pl.ds(hf * hh, hh)],
                dst_ref=qb_rem.at[bi, pl.ds(hf * hh, hh)],
                send_sem=qs_sem.at[j], recv_sem=qr_sem.at[j],
                device_id=ynbr, device_id_type=pl.DeviceIdType.MESH)
            r.start()
            r_q.append(r)

        def partial_attn(src, dst, bi, hf):
            for hj in range(hh):
                hi = hf * hh + hj
                q = src[bi, hi] * scale
                sc = lax.dot_general(q, kt[bi, hi], (((1,), (1,)), ((), ())),
                                     preferred_element_type=jnp.float32)
                m = jnp.max(sc, axis=1, keepdims=True)
                e = jnp.exp(sc - m)
                l = jnp.sum(e, axis=1, keepdims=True)
                o = lax.dot_general(e, vt[bi, hi], (((1,), (0,)), ((), ())),
                                    preferred_element_type=jnp.float32)
                dst[bi, hi, pl.ds(0, sb), :] = o
                dst[bi, hi, sb, :] = m[:, 0]
                dst[bi, hi, sb + 1, :] = l[:, 0]

        for bi, hf in subs:
            partial_attn(qb_send, pl_loc, bi, hf)

        r_o = []
        for j, (bi, hf) in enumerate(subs):
            r_q[j].wait_recv()
            partial_attn(qb_rem, pl_send, bi, hf)
            r = pltpu.make_async_remote_copy(
                src_ref=pl_send.at[bi, pl.ds(hf * hh, hh)],
                dst_ref=pl_rem.at[bi, pl.ds(hf * hh, hh)],
                send_sem=os_sem.at[j], recv_sem=or_sem.at[j],
                device_id=ynbr, device_id_type=pl.DeviceIdType.MESH)
            r.start()
            r_o.append(r)

        r_d = []
        for j, (bi, hf) in enumerate(subs):
            r_o[j].wait_recv()
            for hj in range(hh):
                hi = hf * hh + hj
                m1 = pl_loc[bi, hi, sb, :].reshape(sb, 1)
                l1 = pl_loc[bi, hi, sb + 1, :].reshape(sb, 1)
                m2 = pl_rem[bi, hi, sb, :].reshape(sb, 1)
                l2 = pl_rem[bi, hi, sb + 1, :].reshape(sb, 1)
                mm = jnp.maximum(m1, m2)
                a1 = jnp.exp(m1 - mm)
                a2 = jnp.exp(m2 - mm)
                res = ((a1 * pl_loc[bi, hi, pl.ds(0, sb), :]
                        + a2 * pl_rem[bi, hi, pl.ds(0, sb), :])
                       / (a1 * l1 + a2 * l2))
                fb[bi, hi] = res
                out_t[bi, hi, pl.ds(qoff, sb), :] = res
            for t, nbr in enumerate((xnbr, znbr, dnbr)):
                r = pltpu.make_async_remote_copy(
                    src_ref=fb.at[bi, pl.ds(hf * hh, hh)],
                    dst_ref=out_t.at[bi, pl.ds(hf * hh, hh), pl.ds(qoff, sb), :],
                    send_sem=ds_sem.at[t * NSUB + j],
                    recv_sem=dr_sem.at[t * NSUB + j],
                    device_id=nbr, device_id_type=pl.DeviceIdType.MESH)
                r.start()
                r_d.append(r)

        for j in range(NSUB):
            r_q[j].wait_send()
            r_o[j].wait_send()
        for r in r_d:
            r.wait()

    out_t = pl.pallas_call(
        body,
        out_shape=jax.ShapeDtypeStruct((b, h, s, d), jnp.float32),
        in_specs=[pl.BlockSpec(memory_space=pltpu.VMEM)] * 3,
        out_specs=pl.BlockSpec(memory_space=pltpu.VMEM),
        scratch_shapes=[
            pltpu.VMEM((b, h, sb, d), jnp.float32),
            pltpu.VMEM((b, h, sb, d), jnp.float32),
            pltpu.VMEM((b, h, sb + 2, d), jnp.float32),
            pltpu.VMEM((b, h, sb + 2, d), jnp.float32),
            pltpu.VMEM((b, h, sb + 2, d), jnp.float32),
            pltpu.VMEM((b, h, sb, d), jnp.float32),
            pltpu.SemaphoreType.DMA((NSUB,)),
            pltpu.SemaphoreType.DMA((NSUB,)),
            pltpu.SemaphoreType.DMA((NSUB,)),
            pltpu.SemaphoreType.DMA((NSUB,)),
            pltpu.SemaphoreType.DMA((3 * NSUB,)),
            pltpu.SemaphoreType.DMA((3 * NSUB,)),
        ],
        compiler_params=pltpu.CompilerParams(collective_id=0),
    )(Qt, Kt, Vt)
    return jnp.transpose(out_t, (0, 2, 1, 3))


# device time: 25207 ns/iter; 1.6703x vs baseline; 1.2591x over previous
import jax
import jax.numpy as jnp
from jax import lax
from jax.experimental import pallas as pl
from jax.experimental.pallas import tpu as pltpu

NSUB = 4


def kernel(Q, K, V):
    b, s, h, d = Q.shape
    scale = d ** -0.5
    sb = s // 4
    hh = h // 2
    f32 = jnp.float32
    bf16 = jnp.bfloat16

    Qt = jnp.transpose(Q, (0, 2, 1, 3))
    Kt = jnp.transpose(K, (0, 2, 1, 3))
    Vt = jnp.transpose(V, (0, 2, 1, 3))

    subs = [(bi, hf) for bi in range(b) for hf in range(2)]

    def body(qt, kt, vt, out_t,
             kb, vb, qb_send, qb_rem, pl_loc, pl_send, pl_rem, fb, rcv,
             qs_sem, qr_sem, os_sem, or_sem, ds_sem, dr_sem):
        mx = lax.axis_index("x")
        my = lax.axis_index("y")
        mz = lax.axis_index("z")
        ynbr = (mx, 1 - my, mz)
        xnbr = (1 - mx, my, mz)
        znbr = (mx, my, 1 - mz)
        dnbr = (1 - mx, my, 1 - mz)
        qoff = sb * (2 * mx + mz)

        barrier_sem = pltpu.get_barrier_semaphore()
        for nbr in (ynbr, xnbr, znbr, dnbr):
            pl.semaphore_signal(barrier_sem, inc=1, device_id=nbr,
                                device_id_type=pl.DeviceIdType.MESH)
        pl.semaphore_wait(barrier_sem, 4)

        for bi in range(b):
            for hi in range(h):
                qb_send[bi, hi] = qt[bi, hi, pl.ds(qoff, sb), :].astype(bf16)

        r_q = []
        for j, (bi, hf) in enumerate(subs):
            r = pltpu.make_async_remote_copy(
                src_ref=qb_send.at[bi, pl.ds(hf * hh, hh)],
                dst_ref=qb_rem.at[bi, pl.ds(hf * hh, hh)],
                send_sem=qs_sem.at[j], recv_sem=qr_sem.at[j],
                device_id=ynbr, device_id_type=pl.DeviceIdType.MESH)
            r.start()
            r_q.append(r)

        kb[...] = kt[...].astype(bf16)
        vb[...] = vt[...].astype(bf16)

        def partial_attn(src, dst, bi, hf):
            for hj in range(hh):
                hi = hf * hh + hj
                q = src[bi, hi]
                sc = lax.dot_general(q, kb[bi, hi], (((1,), (1,)), ((), ())),
                                     preferred_element_type=f32) * scale
                m16 = jnp.max(sc, axis=1, keepdims=True).astype(bf16)
                e = jnp.exp(sc - m16.astype(f32))
                l = jnp.sum(e, axis=1, keepdims=True)
                o = lax.dot_general(e.astype(bf16), vb[bi, hi],
                                    (((1,), (0,)), ((), ())),
                                    preferred_element_type=f32)
                dst[bi, hi, pl.ds(0, sb), :] = o.astype(bf16)
                dst[bi, hi, sb, :] = m16[:, 0]
                dst[bi, hi, sb + 1, :] = l.astype(bf16)[:, 0]

        for bi, hf in subs:
            partial_attn(qb_send, pl_loc, bi, hf)

        r_o = []
        for j, (bi, hf) in enumerate(subs):
            r_q[j].wait_recv()
            partial_attn(qb_rem, pl_send, bi, hf)
            r = pltpu.make_async_remote_copy(
                src_ref=pl_send.at[bi, pl.ds(hf * hh, hh)],
                dst_ref=pl_rem.at[bi, pl.ds(hf * hh, hh)],
                send_sem=os_sem.at[j], recv_sem=or_sem.at[j],
                device_id=ynbr, device_id_type=pl.DeviceIdType.MESH)
            r.start()
            r_o.append(r)

        r_d = []
        for j, (bi, hf) in enumerate(subs):
            r_o[j].wait_recv()
            for hj in range(hh):
                hi = hf * hh + hj
                m1 = pl_loc[bi, hi, sb, :].astype(f32).reshape(sb, 1)
                l1 = pl_loc[bi, hi, sb + 1, :].astype(f32).reshape(sb, 1)
                m2 = pl_rem[bi, hi, sb, :].astype(f32).reshape(sb, 1)
                l2 = pl_rem[bi, hi, sb + 1, :].astype(f32).reshape(sb, 1)
                mm = jnp.maximum(m1, m2)
                a1 = jnp.exp(m1 - mm)
                a2 = jnp.exp(m2 - mm)
                o1 = pl_loc[bi, hi, pl.ds(0, sb), :].astype(f32)
                o2 = pl_rem[bi, hi, pl.ds(0, sb), :].astype(f32)
                res = (a1 * o1 + a2 * o2) / (a1 * l1 + a2 * l2)
                fb[bi, hi] = res.astype(bf16)
                out_t[bi, hi, pl.ds(qoff, sb), :] = res
            for t, nbr in enumerate((xnbr, znbr, dnbr)):
                r = pltpu.make_async_remote_copy(
                    src_ref=fb.at[bi, pl.ds(hf * hh, hh)],
                    dst_ref=rcv.at[t, bi, pl.ds(hf * hh, hh)],
                    send_sem=ds_sem.at[t * NSUB + j],
                    recv_sem=dr_sem.at[t * NSUB + j],
                    device_id=nbr, device_id_type=pl.DeviceIdType.MESH)
                r.start()
                r_d.append(r)

        for j in range(NSUB):
            r_q[j].wait_send()
            r_o[j].wait_send()
        for r in r_d:
            r.wait()

        px = sb * (2 * (1 - mx) + mz)
        pz = sb * (2 * mx + (1 - mz))
        pd = sb * (2 * (1 - mx) + (1 - mz))
        for t, off in enumerate((px, pz, pd)):
            for bi in range(b):
                for hi in range(h):
                    out_t[bi, hi, pl.ds(off, sb), :] = rcv[t, bi, hi].astype(f32)

    out_t = pl.pallas_call(
        body,
        out_shape=jax.ShapeDtypeStruct((b, h, s, d), jnp.float32),
        in_specs=[pl.BlockSpec(memory_space=pltpu.VMEM)] * 3,
        out_specs=pl.BlockSpec(memory_space=pltpu.VMEM),
        scratch_shapes=[
            pltpu.VMEM((b, h, s, d), bf16),
            pltpu.VMEM((b, h, s, d), bf16),
            pltpu.VMEM((b, h, sb, d), bf16),
            pltpu.VMEM((b, h, sb, d), bf16),
            pltpu.VMEM((b, h, sb + 2, d), bf16),
            pltpu.VMEM((b, h, sb + 2, d), bf16),
            pltpu.VMEM((b, h, sb + 2, d), bf16),
            pltpu.VMEM((b, h, sb, d), bf16),
            pltpu.VMEM((3, b, h, sb, d), bf16),
            pltpu.SemaphoreType.DMA((NSUB,)),
            pltpu.SemaphoreType.DMA((NSUB,)),
            pltpu.SemaphoreType.DMA((NSUB,)),
            pltpu.SemaphoreType.DMA((NSUB,)),
            pltpu.SemaphoreType.DMA((3 * NSUB,)),
            pltpu.SemaphoreType.DMA((3 * NSUB,)),
        ],
        compiler_params=pltpu.CompilerParams(collective_id=0),
    )(Qt, Kt, Vt)
    return jnp.transpose(out_t, (0, 2, 1, 3))
